# Initial kernel scaffold; baseline (speedup 1.0000x reference)
import jax
import jax.numpy as jnp
from jax.experimental import pallas as pl


def kernel(desc_nir, desc_rgb):
    raise NotImplementedError("write your pallas kernel here")


# speedup vs baseline: 396.9846x; 396.9846x over previous
import jax
import jax.numpy as jnp
from jax.experimental import pallas as pl
from jax.experimental.pallas import tpu as pltpu

NNDR_R = 0.8
KBLK = 2048
PAD_VAL = 1000.0


def _nndr_kernel(q_ref, kt_ref, out_ref, m1_ref, m2_ref):
    i = pl.program_id(0)
    nblk = pl.num_programs(0)

    @pl.when(i == 0)
    def _init():
        m1_ref[...] = jnp.full(m1_ref.shape, jnp.inf, m1_ref.dtype)
        m2_ref[...] = jnp.full(m2_ref.shape, jnp.inf, m2_ref.dtype)

    q = q_ref[...]
    kt = kt_ref[...]
    kn = jnp.sum(kt * kt, axis=0, keepdims=True)
    dot = jnp.dot(q, kt, preferred_element_type=jnp.float32)
    t = kn - 2.0 * dot

    bm1 = jnp.min(t, axis=1, keepdims=True)
    eq = t == bm1
    nmin = jnp.sum(eq.astype(jnp.float32), axis=1, keepdims=True)
    m2cand = jnp.min(jnp.where(eq, jnp.inf, t), axis=1, keepdims=True)
    bm2 = jnp.where(nmin > 1.0, bm1, m2cand)

    m1 = m1_ref[...]
    m2 = m2_ref[...]
    new_m1 = jnp.minimum(m1, bm1)
    new_m2 = jnp.minimum(jnp.maximum(m1, bm1), jnp.minimum(m2, bm2))
    m1_ref[...] = new_m1
    m2_ref[...] = new_m2

    @pl.when(i == nblk - 1)
    def _final():
        qn = jnp.sum(q * q, axis=1, keepdims=True)
        s1 = jnp.maximum(new_m1 + qn, 0.0)
        s2 = jnp.maximum(new_m2 + qn, 0.0)
        d1 = jnp.sqrt(s1)
        d2 = jnp.sqrt(s2)
        mask = d1 < NNDR_R * d2
        per = jnp.sqrt(s1 + 1e-12)
        cnt = jnp.sum(mask.astype(jnp.float32))
        tot = jnp.sum(jnp.where(mask, per, 0.0))
        loss = jnp.where(cnt > 0.0, tot / jnp.maximum(cnt, 1.0), 0.0)
        out_ref[...] = loss.reshape(1, 1)


def kernel(desc_nir, desc_rgb):
    q_n, d = desc_nir.shape
    k_n = desc_rgb.shape[0]
    nblk = pl.cdiv(k_n, KBLK)
    kp = nblk * KBLK
    if kp != k_n:
        pad = jnp.full((kp - k_n, d), PAD_VAL, desc_rgb.dtype)
        desc_rgb = jnp.concatenate([desc_rgb, pad], axis=0)
    kt = desc_rgb.T
    out = pl.pallas_call(
        _nndr_kernel,
        grid=(nblk,),
        in_specs=[pl.BlockSpec((q_n, d), lambda i: (0, 0)),
                  pl.BlockSpec((d, KBLK), lambda i: (0, i))],
        out_specs=pl.BlockSpec((1, 1), lambda i: (0, 0)),
        out_shape=jax.ShapeDtypeStruct((1, 1), jnp.float32),
        scratch_shapes=[pltpu.VMEM((q_n, 1), jnp.float32),
                        pltpu.VMEM((q_n, 1), jnp.float32)],
    )(desc_nir, kt)
    return out[0, 0]


# speedup vs baseline: 500.7084x; 1.2613x over previous
import jax
import jax.numpy as jnp
from jax.experimental import pallas as pl
from jax.experimental.pallas import tpu as pltpu

NNDR_R = 0.8
KBLK = 2048
TOURN_STOP = 128
PAD_VAL = 1000.0


def _nndr_kernel(qa_ref, ka_ref, out_ref, m1_ref, m2_ref):
    i = pl.program_id(0)
    nblk = pl.num_programs(0)

    @pl.when(i == 0)
    def _init():
        m1_ref[...] = jnp.full(m1_ref.shape, jnp.inf, m1_ref.dtype)
        m2_ref[...] = jnp.full(m2_ref.shape, jnp.inf, m2_ref.dtype)

    qa = qa_ref[...]
    ka = ka_ref[...]
    t = jnp.dot(qa, ka, preferred_element_type=jnp.float32)

    w = t.shape[1] // 2
    lo = jnp.minimum(t[:, :w], t[:, w:])
    hi = jnp.maximum(t[:, :w], t[:, w:])
    while w > TOURN_STOP:
        w //= 2
        ll, lr = lo[:, :w], lo[:, w:]
        hl, hr = hi[:, :w], hi[:, w:]
        lo = jnp.minimum(ll, lr)
        hi = jnp.minimum(jnp.maximum(ll, lr), jnp.minimum(hl, hr))

    bm1 = jnp.min(lo, axis=1, keepdims=True)
    eq = lo == bm1
    nmin = jnp.sum(eq.astype(jnp.float32), axis=1, keepdims=True)
    lo2 = jnp.min(jnp.where(eq, jnp.inf, lo), axis=1, keepdims=True)
    hi1 = jnp.min(jnp.where(eq, hi, jnp.inf), axis=1, keepdims=True)
    bm2 = jnp.where(nmin > 1.0, bm1, jnp.minimum(lo2, hi1))

    m1 = m1_ref[...]
    m2 = m2_ref[...]
    new_m1 = jnp.minimum(m1, bm1)
    new_m2 = jnp.minimum(jnp.maximum(m1, bm1), jnp.minimum(m2, bm2))
    m1_ref[...] = new_m1
    m2_ref[...] = new_m2

    @pl.when(i == nblk - 1)
    def _final():
        q2 = qa[:, :-1]
        qn = 0.25 * jnp.sum(q2 * q2, axis=1, keepdims=True)
        s1 = jnp.maximum(new_m1 + qn, 0.0)
        s2 = jnp.maximum(new_m2 + qn, 0.0)
        d1 = jnp.sqrt(s1)
        d2 = jnp.sqrt(s2)
        mask = d1 < NNDR_R * d2
        per = jnp.sqrt(s1 + 1e-12)
        cnt = jnp.sum(mask.astype(jnp.float32))
        tot = jnp.sum(jnp.where(mask, per, 0.0))
        loss = jnp.where(cnt > 0.0, tot / jnp.maximum(cnt, 1.0), 0.0)
        out_ref[...] = loss.reshape(1, 1)


def kernel(desc_nir, desc_rgb):
    q_n, d = desc_nir.shape
    k_n = desc_rgb.shape[0]
    nblk = pl.cdiv(k_n, KBLK)
    kp = nblk * KBLK
    if kp != k_n:
        pad = jnp.full((kp - k_n, d), PAD_VAL, desc_rgb.dtype)
        desc_rgb = jnp.concatenate([desc_rgb, pad], axis=0)
    kn = jnp.sum(desc_rgb * desc_rgb, axis=1)[None, :]
    ka = jnp.concatenate([desc_rgb.T, kn], axis=0)
    qa = jnp.concatenate(
        [-2.0 * desc_nir, jnp.ones((q_n, 1), desc_nir.dtype)], axis=1)
    out = pl.pallas_call(
        _nndr_kernel,
        grid=(nblk,),
        in_specs=[pl.BlockSpec((q_n, d + 1), lambda i: (0, 0)),
                  pl.BlockSpec((d + 1, KBLK), lambda i: (0, i))],
        out_specs=pl.BlockSpec((1, 1), lambda i: (0, 0)),
        out_shape=jax.ShapeDtypeStruct((1, 1), jnp.float32),
        scratch_shapes=[pltpu.VMEM((q_n, 1), jnp.float32),
                        pltpu.VMEM((q_n, 1), jnp.float32)],
    )(qa, ka)
    return out[0, 0]


# speedup vs baseline: 538.9224x; 1.0763x over previous
import jax
import jax.numpy as jnp
from jax.experimental import pallas as pl
from jax.experimental.pallas import tpu as pltpu

NNDR_R = 0.8
KBLK = 4096
TOURN_STOP = 128
PAD_VAL = 1000.0


def _nndr_kernel(qa_ref, ka_ref, out_ref, m1_ref, m2_ref):
    i = pl.program_id(0)
    nblk = pl.num_programs(0)

    @pl.when(i == 0)
    def _init():
        m1_ref[...] = jnp.full(m1_ref.shape, jnp.inf, m1_ref.dtype)
        m2_ref[...] = jnp.full(m2_ref.shape, jnp.inf, m2_ref.dtype)

    qa = qa_ref[...]
    ka = ka_ref[...]
    t = jnp.dot(qa, ka, preferred_element_type=jnp.float32)

    w = t.shape[1] // 2
    lo = jnp.minimum(t[:, :w], t[:, w:])
    hi = jnp.maximum(t[:, :w], t[:, w:])
    while w > TOURN_STOP:
        w //= 2
        ll, lr = lo[:, :w], lo[:, w:]
        hl, hr = hi[:, :w], hi[:, w:]
        lo = jnp.minimum(ll, lr)
        hi = jnp.minimum(jnp.maximum(ll, lr), jnp.minimum(hl, hr))

    bm1 = jnp.min(lo, axis=1, keepdims=True)
    eq = lo == bm1
    nmin = jnp.sum(eq.astype(jnp.float32), axis=1, keepdims=True)
    lo2 = jnp.min(jnp.where(eq, jnp.inf, lo), axis=1, keepdims=True)
    hi1 = jnp.min(jnp.where(eq, hi, jnp.inf), axis=1, keepdims=True)
    bm2 = jnp.where(nmin > 1.0, bm1, jnp.minimum(lo2, hi1))

    m1 = m1_ref[...]
    m2 = m2_ref[...]
    new_m1 = jnp.minimum(m1, bm1)
    new_m2 = jnp.minimum(jnp.maximum(m1, bm1), jnp.minimum(m2, bm2))
    m1_ref[...] = new_m1
    m2_ref[...] = new_m2

    @pl.when(i == nblk - 1)
    def _final():
        q2 = qa[:, :-1]
        qn = 0.25 * jnp.sum(q2 * q2, axis=1, keepdims=True)
        s1 = jnp.maximum(new_m1 + qn, 0.0)
        s2 = jnp.maximum(new_m2 + qn, 0.0)
        d1 = jnp.sqrt(s1)
        d2 = jnp.sqrt(s2)
        mask = d1 < NNDR_R * d2
        per = jnp.sqrt(s1 + 1e-12)
        cnt = jnp.sum(mask.astype(jnp.float32))
        tot = jnp.sum(jnp.where(mask, per, 0.0))
        loss = jnp.where(cnt > 0.0, tot / jnp.maximum(cnt, 1.0), 0.0)
        out_ref[...] = loss.reshape(1, 1)


def kernel(desc_nir, desc_rgb):
    q_n, d = desc_nir.shape
    k_n = desc_rgb.shape[0]
    nblk = pl.cdiv(k_n, KBLK)
    kp = nblk * KBLK
    if kp != k_n:
        pad = jnp.full((kp - k_n, d), PAD_VAL, desc_rgb.dtype)
        desc_rgb = jnp.concatenate([desc_rgb, pad], axis=0)
    kn = jnp.sum(desc_rgb * desc_rgb, axis=1)[None, :]
    ka = jnp.concatenate([desc_rgb.T, kn], axis=0)
    qa = jnp.concatenate(
        [-2.0 * desc_nir, jnp.ones((q_n, 1), desc_nir.dtype)], axis=1)
    out = pl.pallas_call(
        _nndr_kernel,
        grid=(nblk,),
        in_specs=[pl.BlockSpec((q_n, d + 1), lambda i: (0, 0)),
                  pl.BlockSpec((d + 1, KBLK), lambda i: (0, i))],
        out_specs=pl.BlockSpec((1, 1), lambda i: (0, 0)),
        out_shape=jax.ShapeDtypeStruct((1, 1), jnp.float32),
        scratch_shapes=[pltpu.VMEM((q_n, 1), jnp.float32),
                        pltpu.VMEM((q_n, 1), jnp.float32)],
    )(qa, ka)
    return out[0, 0]


# speedup vs baseline: 566.9219x; 1.0520x over previous
import jax
import jax.numpy as jnp
from jax.experimental import pallas as pl
from jax.experimental.pallas import tpu as pltpu

NNDR_R = 0.8
KBLK = 4096
STATE_W = 128
PAD_VAL = 1000.0


def _pair_merge(ll, hl, lr, hr):
    lo = jnp.minimum(ll, lr)
    hi = jnp.minimum(jnp.maximum(ll, lr), jnp.minimum(hl, hr))
    return lo, hi


def _nndr_kernel(qs_ref, ka_ref, out_ref, lo_ref, hi_ref):
    i = pl.program_id(0)
    nblk = pl.num_programs(0)

    @pl.when(i == 0)
    def _init():
        lo_ref[...] = jnp.full(lo_ref.shape, jnp.inf, lo_ref.dtype)
        hi_ref[...] = jnp.full(hi_ref.shape, jnp.inf, hi_ref.dtype)

    qs = qs_ref[...]
    kt = ka_ref[:-1, :]
    kn = ka_ref[-1:, :]
    dot2 = jnp.dot(qs, kt, preferred_element_type=jnp.float32)
    t = kn + dot2

    w = t.shape[1] // 2
    lo = jnp.minimum(t[:, :w], t[:, w:])
    hi = jnp.maximum(t[:, :w], t[:, w:])
    while w > STATE_W:
        w //= 2
        lo, hi = _pair_merge(lo[:, :w], hi[:, :w], lo[:, w:], hi[:, w:])

    lo, hi = _pair_merge(lo_ref[...], hi_ref[...], lo, hi)
    lo_ref[...] = lo
    hi_ref[...] = hi

    @pl.when(i == nblk - 1)
    def _final():
        flo, fhi = lo, hi
        fw = STATE_W
        while fw > 8:
            fw //= 2
            flo, fhi = _pair_merge(flo[:, :fw], fhi[:, :fw],
                                   flo[:, fw:], fhi[:, fw:])
        bm1 = jnp.min(flo, axis=1, keepdims=True)
        eq = flo == bm1
        nmin = jnp.sum(eq.astype(jnp.float32), axis=1, keepdims=True)
        lo2 = jnp.min(jnp.where(eq, jnp.inf, flo), axis=1, keepdims=True)
        hi1 = jnp.min(jnp.where(eq, fhi, jnp.inf), axis=1, keepdims=True)
        bm2 = jnp.where(nmin > 1.0, bm1, jnp.minimum(lo2, hi1))

        qn = 0.25 * jnp.sum(qs * qs, axis=1, keepdims=True)
        s1 = jnp.maximum(bm1 + qn, 0.0)
        s2 = jnp.maximum(bm2 + qn, 0.0)
        d1 = jnp.sqrt(s1)
        d2 = jnp.sqrt(s2)
        mask = d1 < NNDR_R * d2
        per = jnp.sqrt(s1 + 1e-12)
        cnt = jnp.sum(mask.astype(jnp.float32))
        tot = jnp.sum(jnp.where(mask, per, 0.0))
        loss = jnp.where(cnt > 0.0, tot / jnp.maximum(cnt, 1.0), 0.0)
        out_ref[...] = loss.reshape(1, 1)


def kernel(desc_nir, desc_rgb):
    q_n, d = desc_nir.shape
    k_n = desc_rgb.shape[0]
    nblk = pl.cdiv(k_n, KBLK)
    kp = nblk * KBLK
    if kp != k_n:
        pad = jnp.full((kp - k_n, d), PAD_VAL, desc_rgb.dtype)
        desc_rgb = jnp.concatenate([desc_rgb, pad], axis=0)
    kn = jnp.sum(desc_rgb * desc_rgb, axis=1)[None, :]
    ka = jnp.concatenate([desc_rgb.T, kn], axis=0)
    out = pl.pallas_call(
        _nndr_kernel,
        grid=(nblk,),
        in_specs=[pl.BlockSpec((q_n, d), lambda i: (0, 0)),
                  pl.BlockSpec((d + 1, KBLK), lambda i: (0, i))],
        out_specs=pl.BlockSpec((1, 1), lambda i: (0, 0)),
        out_shape=jax.ShapeDtypeStruct((1, 1), jnp.float32),
        scratch_shapes=[pltpu.VMEM((q_n, STATE_W), jnp.float32),
                        pltpu.VMEM((q_n, STATE_W), jnp.float32)],
    )(-2.0 * desc_nir, ka)
    return out[0, 0]
